# baseline (speedup 1.0000x reference)
import functools

import jax
import jax.numpy as jnp
from jax import lax
from jax.experimental import pallas as pl
from jax.experimental.pallas import tpu as pltpu
from jax.experimental.pallas import tpu_sc as plsc

NUM_LEVELS = 5
C = 8
LEVELS = (16, 32, 64, 128, 256)
B = 1048576
NCORES = 2
NSUB = 16
NW = NCORES * NSUB
NPW = B // NW
NC = 256
NCHUNKS = NPW // NC
OUTW = NUM_LEVELS * C
PW = 2 * C


def _tec_body(x_h, y_h, z_h, t0, t1, t2, t3, t4, out,
              x_b, y_b, z_b, u_b, v_b, valid_b,
              fp0, fp1, fp2, fp3, fp4, fq0, fq1, fq2, fq3, fq4,
              face_b, i0, i1, i2, i3, i4, j0, j1, j2, j3, j4,
              gt0, gt1, gt2, gt3, gt4, gb0, gb1, gb2, gb3, gb4, outc,
              sa0, sa1, sa2, sa3, sa4, sb0, sb1, sb2, sb3, sb4,
              sc0, sc1, sc2, sc3, sc4, sd0, sd1, sd2, sd3, sd4):
    tabs = (t0, t1, t2, t3, t4)
    fps = (fp0, fp1, fp2, fp3, fp4)
    fqs = (fq0, fq1, fq2, fq3, fq4)
    ibs = (i0, i1, i2, i3, i4)
    jbs = (j0, j1, j2, j3, j4)
    gts = (gt0, gt1, gt2, gt3, gt4)
    gbs = (gb0, gb1, gb2, gb3, gb4)
    sems = ((sa0, sa1, sa2, sa3, sa4), (sb0, sb1, sb2, sb3, sb4),
            (sc0, sc1, sc2, sc3, sc4), (sd0, sd1, sd2, sd3, sd4))
    wid = lax.axis_index("s") * NCORES + lax.axis_index("c")
    base = wid * NPW

    iota = lax.iota(jnp.int32, 16)
    hif = (iota >> 3).astype(jnp.float32)
    sgn = 1.0 - 2.0 * hif
    rot = (iota ^ 8).reshape(16, 1)
    rot_dnums = lax.GatherDimensionNumbers(
        offset_dims=(), collapsed_slice_dims=(0,), start_index_map=(0,))

    def do_ab(cb, st):
        xb_v = x_b.at[st]
        yb_v = y_b.at[st]
        zb_v = z_b.at[st]
        ub_v = u_b.at[st]
        vb_v = v_b.at[st]
        vm_v = valid_b.at[st]
        fc_v = face_b.at[st]
        pltpu.sync_copy(x_h.at[pl.ds(cb, NC)], xb_v)
        pltpu.sync_copy(y_h.at[pl.ds(cb, NC)], yb_v)
        pltpu.sync_copy(z_h.at[pl.ds(cb, NC)], zb_v)

        def pass_a(j, c):
            s = j * 16
            x = xb_v[pl.ds(s, 16)]
            y = yb_v[pl.ds(s, 16)]
            z = zb_v[pl.ds(s, 16)]
            ax, ay, az = jnp.abs(x), jnp.abs(y), jnp.abs(z)
            ma = jnp.maximum(jnp.maximum(ax, ay), az)
            gxy = jnp.where(ax >= ay, 1.0, 0.0)
            gxz = jnp.where(ax >= az, 1.0, 0.0)
            gyz = jnp.where(ay >= az, 1.0, 0.0)
            xp = jnp.where(x > 0.0, 1.0, 0.0)
            yp = jnp.where(y > 0.0, 1.0, 0.0)
            zp = jnp.where(z > 0.0, 1.0, 0.0)
            fx = gxy * gxz
            fy = (1.0 - fx) * gyz
            fz = 1.0 - fx - fy
            face_f = fx * (1.0 - xp) + fy * (3.0 - yp) + fz * (5.0 - zp)
            sc = fx * z * (1.0 - 2.0 * xp) + fy * x + fz * x * (2.0 * zp - 1.0)
            tc = fy * z * (2.0 * yp - 1.0) - (1.0 - fy) * y
            vm = jnp.where(ma > 0.0, 1.0, 0.0)
            inv = 1.0 / (ma + (1.0 - vm))
            ub_v[pl.ds(s, 16)] = 0.5 * (sc * inv + 1.0)
            vb_v[pl.ds(s, 16)] = 0.5 * (tc * inv + 1.0)
            fc_v[pl.ds(s, 16)] = face_f.astype(jnp.int32)
            vm_v[pl.ds(s, 16)] = vm
            return c

        lax.fori_loop(0, NC // 16, pass_a, 0)

        for lvl in range(NUM_LEVELS):
            L = LEVELS[lvl]
            fvrow = 6 * L * L
            fp_v = fps[lvl].at[st]
            fq_v = fqs[lvl].at[st]
            ib_v = ibs[lvl].at[st]
            jb_v = jbs[lvl].at[st]

            def pass_b(j, c, L=L, fvrow=fvrow, fp_v=fp_v, fq_v=fq_v,
                       ib_v=ib_v, jb_v=jb_v):
                s = j * 16
                face = fc_v[pl.ds(s, 16)]
                vm = vm_v[pl.ds(s, 16)]
                p = ub_v[pl.ds(s, 16)] * float(L) - 0.5
                q = vb_v[pl.ds(s, 16)] * float(L) - 0.5
                p0 = (p + 1.0).astype(jnp.int32) - 1
                q0 = (q + 1.0).astype(jnp.int32) - 1
                xb = jnp.maximum(jnp.minimum(p0, L - 2), 0)
                yb = jnp.maximum(jnp.minimum(q0, L - 2), 0)
                fp = p - xb.astype(jnp.float32)
                fq = q - yb.astype(jnp.float32)
                fp_v[pl.ds(s, 16)] = jnp.maximum(jnp.minimum(fp, 1.0), 0.0)
                fq_v[pl.ds(s, 16)] = jnp.maximum(jnp.minimum(fq, 1.0), 0.0)
                r0 = face * (L * L) + yb * L + xb
                rt = jnp.where(vm > 0.5, r0, fvrow)
                ib_v[pl.ds(s, 16)] = rt
                jb_v[pl.ds(s, 16)] = rt + L
                return c

            lax.fori_loop(0, NC // 16, pass_b, 0)
            pltpu.async_copy(tabs[lvl].at[ib_v], gts[lvl].at[st],
                             sems[2 * st][lvl])
            pltpu.async_copy(tabs[lvl].at[jb_v], gbs[lvl].at[st],
                             sems[2 * st + 1][lvl])

    def wait_set(st):
        for lvl in range(NUM_LEVELS):
            pltpu.make_async_copy(tabs[lvl].at[ibs[lvl].at[st]],
                                  gts[lvl].at[st], sems[2 * st][lvl]).wait()
            pltpu.make_async_copy(tabs[lvl].at[jbs[lvl].at[st]],
                                  gbs[lvl].at[st], sems[2 * st + 1][lvl]).wait()

    def do_c(cb, st):
        wait_set(st)
        for lvl in range(NUM_LEVELS):
            gt = gts[lvl].at[st]
            gb = gbs[lvl].at[st]
            fp_v = fps[lvl].at[st]
            fq_v = fqs[lvl].at[st]

            def pass_c(jo, c, lvl=lvl, gt=gt, gb=gb, fp_v=fp_v, fq_v=fq_v):
                s = jo * 16
                wp16 = fp_v[pl.ds(s, 16)]
                wq16 = fq_v[pl.ds(s, 16)]
                for jj in range(16):
                    j = s + jj
                    tv = gt[j, pl.ds(0, 16)]
                    bv = gb[j, pl.ds(0, 16)]
                    wq = jnp.broadcast_to(wq16[jj], (16,))
                    t = tv + (bv - tv) * wq
                    r = lax.gather(
                        t, rot, rot_dnums, slice_sizes=(1,),
                        mode=lax.GatherScatterMode.PROMISE_IN_BOUNDS)
                    a = jnp.broadcast_to(wp16[jj], (16,))
                    wb = hif + sgn * a
                    o = t + (r - t) * wb
                    outc[j, pl.ds(lvl * C, 16)] = o
                return c

            lax.fori_loop(0, NC // 16, pass_c, 0)

        pltpu.sync_copy(outc.at[:, pl.ds(0, OUTW)],
                        out.at[pl.ds(cb, NC), :])

    do_ab(base, 0)

    def body(k, carry):
        cb0 = base + (2 * k) * NC
        cb1 = cb0 + NC
        cb2 = jnp.minimum(cb0 + 2 * NC, base + NPW - NC)
        do_ab(cb1, 1)
        do_c(cb0, 0)
        do_ab(cb2, 0)
        do_c(cb1, 1)
        return carry

    lax.fori_loop(0, NCHUNKS // 2, body, 0)
    wait_set(0)


def kernel(inputs, params0, params1, params2, params3, params4, fail_value):
    xt = inputs.T
    x_h, y_h, z_h = xt[0], xt[1], xt[2]
    tabs = []
    for prm, L in zip((params0, params1, params2, params3, params4), LEVELS):
        F, Ch, _, _ = prm.shape
        d = jnp.transpose(prm, (0, 2, 3, 1)).reshape(F * L * L, Ch)
        pair = jnp.concatenate([d, jnp.roll(d, -1, axis=0)], axis=1)
        pad = jnp.tile(jnp.concatenate([fail_value, fail_value])[None, :],
                       (L + 2, 1))
        tabs.append(jnp.concatenate([pair, pad], axis=0))

    mesh = plsc.VectorSubcoreMesh(core_axis_name="c", subcore_axis_name="s")
    run = functools.partial(
        pl.kernel,
        mesh=mesh,
        compiler_params=pltpu.CompilerParams(use_tc_tiling_on_sc=False),
        out_type=jax.ShapeDtypeStruct((B, OUTW), jnp.float32),
        scratch_types=[
            pltpu.VMEM((2, NC), jnp.float32),
            pltpu.VMEM((2, NC), jnp.float32),
            pltpu.VMEM((2, NC), jnp.float32),
            pltpu.VMEM((2, NC), jnp.float32),
            pltpu.VMEM((2, NC), jnp.float32),
            pltpu.VMEM((2, NC), jnp.float32),
        ] + [pltpu.VMEM((2, NC), jnp.float32) for _ in range(NUM_LEVELS)]
          + [pltpu.VMEM((2, NC), jnp.float32) for _ in range(NUM_LEVELS)]
          + [pltpu.VMEM((2, NC), jnp.int32)]
          + [pltpu.VMEM((2, NC), jnp.int32) for _ in range(NUM_LEVELS)]
          + [pltpu.VMEM((2, NC), jnp.int32) for _ in range(NUM_LEVELS)]
          + [pltpu.VMEM((2, NC, PW), jnp.float32) for _ in range(NUM_LEVELS)]
          + [pltpu.VMEM((2, NC, PW), jnp.float32) for _ in range(NUM_LEVELS)]
          + [pltpu.VMEM((NC, OUTW + C), jnp.float32)]
          + [pltpu.SemaphoreType.DMA for _ in range(4 * NUM_LEVELS)],
    )(_tec_body)
    return run(x_h, y_h, z_h, tabs[0], tabs[1], tabs[2], tabs[3], tabs[4])


# speedup vs baseline: 1.0186x; 1.0186x over previous
import functools

import jax
import jax.numpy as jnp
from jax import lax
from jax.experimental import pallas as pl
from jax.experimental.pallas import tpu as pltpu
from jax.experimental.pallas import tpu_sc as plsc

NUM_LEVELS = 5
C = 8
LEVELS = (16, 32, 64, 128, 256)
B = 1048576
NCORES = 2
NSUB = 16
NW = NCORES * NSUB
NPW = B // NW
NC = 512
NCHUNKS = NPW // NC
OUTW = NUM_LEVELS * C
PW = 2 * C


def _tec_body(x_h, y_h, z_h, t0, t1, t2, t3, t4, out,
              x_b, y_b, z_b, u_b, v_b, valid_b,
              fp0, fp1, fp2, fp3, fp4, fq0, fq1, fq2, fq3, fq4,
              face_b, i0, i1, i2, i3, i4, j0, j1, j2, j3, j4,
              gt0, gt1, gt2, gt3, gt4, gb0, gb1, gb2, gb3, gb4, outc,
              st0, st1, st2, st3, st4, sb0, sb1, sb2, sb3, sb4):
    tabs = (t0, t1, t2, t3, t4)
    fps = (fp0, fp1, fp2, fp3, fp4)
    fqs = (fq0, fq1, fq2, fq3, fq4)
    ibs = (i0, i1, i2, i3, i4)
    jbs = (j0, j1, j2, j3, j4)
    gts = (gt0, gt1, gt2, gt3, gt4)
    gbs = (gb0, gb1, gb2, gb3, gb4)
    sts = (st0, st1, st2, st3, st4)
    sbs = (sb0, sb1, sb2, sb3, sb4)
    wid = lax.axis_index("s") * NCORES + lax.axis_index("c")
    base = wid * NPW

    iota = lax.iota(jnp.int32, 16)
    hif = (iota >> 3).astype(jnp.float32)
    sgn = 1.0 - 2.0 * hif
    rot = (iota ^ 8).reshape(16, 1)
    rot_dnums = lax.GatherDimensionNumbers(
        offset_dims=(), collapsed_slice_dims=(0,), start_index_map=(0,))

    def chunk_body(ci, carry):
        cb = base + ci * NC
        pltpu.sync_copy(x_h.at[pl.ds(cb, NC)], x_b)
        pltpu.sync_copy(y_h.at[pl.ds(cb, NC)], y_b)
        pltpu.sync_copy(z_h.at[pl.ds(cb, NC)], z_b)

        def pass_a(j, c):
            s = j * 16
            x = x_b[pl.ds(s, 16)]
            y = y_b[pl.ds(s, 16)]
            z = z_b[pl.ds(s, 16)]
            ax, ay, az = jnp.abs(x), jnp.abs(y), jnp.abs(z)
            ma = jnp.maximum(jnp.maximum(ax, ay), az)
            gxy = jnp.where(ax >= ay, 1.0, 0.0)
            gxz = jnp.where(ax >= az, 1.0, 0.0)
            gyz = jnp.where(ay >= az, 1.0, 0.0)
            xp = jnp.where(x > 0.0, 1.0, 0.0)
            yp = jnp.where(y > 0.0, 1.0, 0.0)
            zp = jnp.where(z > 0.0, 1.0, 0.0)
            fx = gxy * gxz
            fy = (1.0 - fx) * gyz
            fz = 1.0 - fx - fy
            face_f = fx * (1.0 - xp) + fy * (3.0 - yp) + fz * (5.0 - zp)
            sc = fx * z * (1.0 - 2.0 * xp) + fy * x + fz * x * (2.0 * zp - 1.0)
            tc = fy * z * (2.0 * yp - 1.0) - (1.0 - fy) * y
            vm = jnp.where(ma > 0.0, 1.0, 0.0)
            inv = 1.0 / (ma + (1.0 - vm))
            u_b[pl.ds(s, 16)] = 0.5 * (sc * inv + 1.0)
            v_b[pl.ds(s, 16)] = 0.5 * (tc * inv + 1.0)
            face_b[pl.ds(s, 16)] = face_f.astype(jnp.int32)
            valid_b[pl.ds(s, 16)] = vm
            return c

        lax.fori_loop(0, NC // 16, pass_a, 0)

        copies = []
        for lvl in range(NUM_LEVELS):
            L = LEVELS[lvl]
            fvrow = 6 * L * L

            def pass_b(j, c, lvl=lvl, L=L, fvrow=fvrow):
                s = j * 16
                face = face_b[pl.ds(s, 16)]
                vm = valid_b[pl.ds(s, 16)]
                p = u_b[pl.ds(s, 16)] * float(L) - 0.5
                q = v_b[pl.ds(s, 16)] * float(L) - 0.5
                p0 = (p + 1.0).astype(jnp.int32) - 1
                q0 = (q + 1.0).astype(jnp.int32) - 1
                xb = jnp.maximum(jnp.minimum(p0, L - 2), 0)
                yb = jnp.maximum(jnp.minimum(q0, L - 2), 0)
                fp = p - xb.astype(jnp.float32)
                fq = q - yb.astype(jnp.float32)
                fps[lvl][pl.ds(s, 16)] = jnp.maximum(jnp.minimum(fp, 1.0), 0.0)
                fqs[lvl][pl.ds(s, 16)] = jnp.maximum(jnp.minimum(fq, 1.0), 0.0)
                r0 = face * (L * L) + yb * L + xb
                rt = jnp.where(vm > 0.5, r0, fvrow)
                ibs[lvl][pl.ds(s, 16)] = rt
                jbs[lvl][pl.ds(s, 16)] = rt + L
                return c

            lax.fori_loop(0, NC // 16, pass_b, 0)
            copies.append(
                pltpu.async_copy(tabs[lvl].at[ibs[lvl]], gts[lvl], sts[lvl]))
            copies.append(
                pltpu.async_copy(tabs[lvl].at[jbs[lvl]], gbs[lvl], sbs[lvl]))

        for lvl in range(NUM_LEVELS):
            copies[2 * lvl].wait()
            copies[2 * lvl + 1].wait()
            gt = gts[lvl]
            gb = gbs[lvl]

            def pass_c(jo, c, lvl=lvl, gt=gt, gb=gb):
                s = jo * 16
                wp16 = fps[lvl][pl.ds(s, 16)]
                wq16 = fqs[lvl][pl.ds(s, 16)]
                for jj in range(16):
                    j = s + jj
                    tv = gt[j, pl.ds(0, 16)]
                    bv = gb[j, pl.ds(0, 16)]
                    wq = jnp.broadcast_to(wq16[jj], (16,))
                    t = tv + (bv - tv) * wq
                    r = lax.gather(
                        t, rot, rot_dnums, slice_sizes=(1,),
                        mode=lax.GatherScatterMode.PROMISE_IN_BOUNDS)
                    a = jnp.broadcast_to(wp16[jj], (16,))
                    wb = hif + sgn * a
                    o = t + (r - t) * wb
                    outc[j, pl.ds(lvl * C, 16)] = o
                return c

            lax.fori_loop(0, NC // 16, pass_c, 0)

        pltpu.sync_copy(outc.at[:, pl.ds(0, OUTW)],
                        out.at[pl.ds(cb, NC), :])
        return carry

    lax.fori_loop(0, NCHUNKS, chunk_body, 0)


def kernel(inputs, params0, params1, params2, params3, params4, fail_value):
    xt = inputs.T
    x_h, y_h, z_h = xt[0], xt[1], xt[2]
    tabs = []
    for prm, L in zip((params0, params1, params2, params3, params4), LEVELS):
        F, Ch, _, _ = prm.shape
        d = jnp.transpose(prm, (0, 2, 3, 1)).reshape(F * L * L, Ch)
        pair = jnp.concatenate([d, jnp.roll(d, -1, axis=0)], axis=1)
        pad = jnp.tile(jnp.concatenate([fail_value, fail_value])[None, :],
                       (L + 2, 1))
        tabs.append(jnp.concatenate([pair, pad], axis=0))

    mesh = plsc.VectorSubcoreMesh(core_axis_name="c", subcore_axis_name="s")
    run = functools.partial(
        pl.kernel,
        mesh=mesh,
        compiler_params=pltpu.CompilerParams(use_tc_tiling_on_sc=False),
        out_type=jax.ShapeDtypeStruct((B, OUTW), jnp.float32),
        scratch_types=[
            pltpu.VMEM((NC,), jnp.float32),
            pltpu.VMEM((NC,), jnp.float32),
            pltpu.VMEM((NC,), jnp.float32),
            pltpu.VMEM((NC,), jnp.float32),
            pltpu.VMEM((NC,), jnp.float32),
            pltpu.VMEM((NC,), jnp.float32),
        ] + [pltpu.VMEM((NC,), jnp.float32) for _ in range(NUM_LEVELS)]
          + [pltpu.VMEM((NC,), jnp.float32) for _ in range(NUM_LEVELS)]
          + [pltpu.VMEM((NC,), jnp.int32)]
          + [pltpu.VMEM((NC,), jnp.int32) for _ in range(NUM_LEVELS)]
          + [pltpu.VMEM((NC,), jnp.int32) for _ in range(NUM_LEVELS)]
          + [pltpu.VMEM((NC, PW), jnp.float32) for _ in range(NUM_LEVELS)]
          + [pltpu.VMEM((NC, PW), jnp.float32) for _ in range(NUM_LEVELS)]
          + [pltpu.VMEM((NC, OUTW + C), jnp.float32)]
          + [pltpu.SemaphoreType.DMA for _ in range(2 * NUM_LEVELS)],
    )(_tec_body)
    return run(x_h, y_h, z_h, tabs[0], tabs[1], tabs[2], tabs[3], tabs[4])
